# Initial kernel scaffold; baseline (speedup 1.0000x reference)
import functools

import jax
import jax.numpy as jnp
from jax.experimental import pallas as pl


def _gate_kernel(x_ref, gw1_ref, gb1_ref, gw2_ref, gb2_ref, gwt_ref, load_ref):
    x = x_ref[...]
    gh = jax.nn.gelu(
        jnp.dot(x, gw1_ref[...], preferred_element_type=jnp.float32) + gb1_ref[...],
        approximate=False,
    )
    l = jnp.dot(gh, gw2_ref[...], preferred_element_type=jnp.float32) + gb2_ref[...]
    e_dim = l.shape[-1]
    col = jax.lax.broadcasted_iota(jnp.int32, l.shape, 1)
    m1 = jnp.max(l, axis=-1, keepdims=True)
    i1 = jnp.min(jnp.where(l == m1, col, e_dim), axis=-1, keepdims=True)
    lm = jnp.where(col == i1, -jnp.inf, l)
    m2 = jnp.max(lm, axis=-1, keepdims=True)
    i2 = jnp.min(jnp.where(lm == m2, col, e_dim), axis=-1, keepdims=True)
    kept = (col == i1) | (col == i2)
    s = jnp.where(kept, l, 0.0)
    mx = jnp.maximum(m1, 0.0)
    ex = jnp.exp(s - mx)
    gw = ex / jnp.sum(ex, axis=-1, keepdims=True)
    gwt_ref[...] = gw

    @pl.when(pl.program_id(0) == 0)
    def _init():
        load_ref[...] = jnp.zeros_like(load_ref)

    load_ref[...] += jnp.sum(gw, axis=0, keepdims=True)


def _moe_kernel(x_ref, w1_ref, b1_ref, w2_ref, b2_ref, gw_ref,
                logits_ref, alpha_ref, *, n_exp):
    e = pl.program_id(1)
    x = x_ref[...]
    h = jax.nn.gelu(
        jnp.dot(x, w1_ref[0], preferred_element_type=jnp.float32) + b1_ref[0],
        approximate=False,
    )
    o = jnp.dot(h, w2_ref[0], preferred_element_type=jnp.float32) + b2_ref[0]

    @pl.when(e < n_exp)
    def _combine():
        col = jax.lax.broadcasted_iota(jnp.int32, gw_ref.shape, 1)
        w = jnp.sum(jnp.where(col == e, gw_ref[...], 0.0), axis=-1, keepdims=True)
        contrib = o * w

        @pl.when(e == 0)
        def _first():
            logits_ref[...] = contrib

        @pl.when(e > 0)
        def _rest():
            logits_ref[...] += contrib

    @pl.when(e == n_exp)
    def _alpha():
        alpha_ref[...] = jax.nn.softplus(o) + 1e-6


def kernel(node_features, gw1, gb1, gw2, gb2, ew1, eb1, ew2, eb2, aw1, ab1, aw2, ab2):
    x = node_features
    n, d = x.shape
    e_num = gw2.shape[1]
    h_dim = ew1.shape[2]
    c_dim = ew2.shape[2]
    tn = min(n, 1024)
    nt = n // tn

    gate_weights, load2d = pl.pallas_call(
        _gate_kernel,
        grid=(nt,),
        in_specs=[
            pl.BlockSpec((tn, d), lambda i: (i, 0)),
            pl.BlockSpec((d, d), lambda i: (0, 0)),
            pl.BlockSpec((1, d), lambda i: (0, 0)),
            pl.BlockSpec((d, e_num), lambda i: (0, 0)),
            pl.BlockSpec((1, e_num), lambda i: (0, 0)),
        ],
        out_specs=[
            pl.BlockSpec((tn, e_num), lambda i: (i, 0)),
            pl.BlockSpec((1, e_num), lambda i: (0, 0)),
        ],
        out_shape=[
            jax.ShapeDtypeStruct((n, e_num), jnp.float32),
            jax.ShapeDtypeStruct((1, e_num), jnp.float32),
        ],
    )(x, gw1, gb1.reshape(1, d), gw2, gb2.reshape(1, e_num))

    w1_all = jnp.concatenate([ew1, aw1[None]], axis=0)
    b1_all = jnp.concatenate([eb1, ab1[None]], axis=0).reshape(e_num + 1, 1, h_dim)
    w2_all = jnp.concatenate([ew2, aw2[None]], axis=0)
    b2_all = jnp.concatenate([eb2, ab2[None]], axis=0).reshape(e_num + 1, 1, c_dim)

    logits, alpha = pl.pallas_call(
        functools.partial(_moe_kernel, n_exp=e_num),
        grid=(nt, e_num + 1),
        in_specs=[
            pl.BlockSpec((tn, d), lambda i, j: (i, 0)),
            pl.BlockSpec((1, d, h_dim), lambda i, j: (j, 0, 0)),
            pl.BlockSpec((1, 1, h_dim), lambda i, j: (j, 0, 0)),
            pl.BlockSpec((1, h_dim, c_dim), lambda i, j: (j, 0, 0)),
            pl.BlockSpec((1, 1, c_dim), lambda i, j: (j, 0, 0)),
            pl.BlockSpec((tn, e_num), lambda i, j: (i, 0)),
        ],
        out_specs=[
            pl.BlockSpec((tn, c_dim), lambda i, j: (i, 0)),
            pl.BlockSpec((tn, c_dim), lambda i, j: (i, 0)),
        ],
        out_shape=[
            jax.ShapeDtypeStruct((n, c_dim), jnp.float32),
            jax.ShapeDtypeStruct((n, c_dim), jnp.float32),
        ],
    )(x, w1_all, b1_all, w2_all, b2_all, gate_weights)

    return (logits, alpha, gate_weights, load2d.reshape(e_num))


# speedup vs baseline: 2.4330x; 2.4330x over previous
import functools

import jax
import jax.numpy as jnp
from jax.experimental import pallas as pl

_SQRT_HALF = 0.7071067811865476


def _gelu(v):
    return 0.5 * v * (1.0 + jax.lax.erf(v * _SQRT_HALF))


def _gate_kernel(x_ref, gw1_ref, gb1_ref, gw2_ref, gb2_ref, gwt_ref, load_ref):
    x = x_ref[...]
    gh = _gelu(jnp.dot(x, gw1_ref[...], preferred_element_type=jnp.float32) + gb1_ref[...])
    l = jnp.dot(gh, gw2_ref[...], preferred_element_type=jnp.float32) + gb2_ref[...]
    e_dim = l.shape[-1]
    col = jax.lax.broadcasted_iota(jnp.int32, l.shape, 1)
    m1 = jnp.max(l, axis=-1, keepdims=True)
    i1 = jnp.min(jnp.where(l == m1, col, e_dim), axis=-1, keepdims=True)
    lm = jnp.where(col == i1, -jnp.inf, l)
    m2 = jnp.max(lm, axis=-1, keepdims=True)
    i2 = jnp.min(jnp.where(lm == m2, col, e_dim), axis=-1, keepdims=True)
    kept = (col == i1) | (col == i2)
    s = jnp.where(kept, l, 0.0)
    mx = jnp.maximum(m1, 0.0)
    ex = jnp.exp(s - mx)
    gw = ex / jnp.sum(ex, axis=-1, keepdims=True)
    gwt_ref[...] = gw

    @pl.when(pl.program_id(0) == 0)
    def _init():
        load_ref[...] = jnp.zeros_like(load_ref)

    load_ref[...] += jnp.sum(gw, axis=0, keepdims=True)


def _moe_kernel(x_ref, w1_ref, b1_ref, w2_ref, b2_ref, gw_ref,
                logits_ref, alpha_ref, *, n_exp):
    e = pl.program_id(1)
    x = x_ref[...]
    h = _gelu(jnp.dot(x, w1_ref[0], preferred_element_type=jnp.float32) + b1_ref[0])
    o = jnp.dot(h.astype(w2_ref.dtype), w2_ref[0],
                preferred_element_type=jnp.float32) + b2_ref[0]

    @pl.when(e < n_exp)
    def _combine():
        col = jax.lax.broadcasted_iota(jnp.int32, gw_ref.shape, 1)
        w = jnp.sum(jnp.where(col == e, gw_ref[...], 0.0), axis=-1, keepdims=True)
        contrib = o * w

        @pl.when(e == 0)
        def _first():
            logits_ref[...] = contrib

        @pl.when(e > 0)
        def _rest():
            logits_ref[...] += contrib

    @pl.when(e == n_exp)
    def _alpha():
        alpha_ref[...] = jax.nn.softplus(o) + 1e-6


def kernel(node_features, gw1, gb1, gw2, gb2, ew1, eb1, ew2, eb2, aw1, ab1, aw2, ab2):
    x = node_features
    n, d = x.shape
    e_num = gw2.shape[1]
    h_dim = ew1.shape[2]
    c_dim = ew2.shape[2]
    tn = min(n, 1024)
    nt = n // tn

    gate_weights, load2d = pl.pallas_call(
        _gate_kernel,
        grid=(nt,),
        in_specs=[
            pl.BlockSpec((tn, d), lambda i: (i, 0)),
            pl.BlockSpec((d, d), lambda i: (0, 0)),
            pl.BlockSpec((1, d), lambda i: (0, 0)),
            pl.BlockSpec((d, e_num), lambda i: (0, 0)),
            pl.BlockSpec((1, e_num), lambda i: (0, 0)),
        ],
        out_specs=[
            pl.BlockSpec((tn, e_num), lambda i: (i, 0)),
            pl.BlockSpec((1, e_num), lambda i: (0, 0)),
        ],
        out_shape=[
            jax.ShapeDtypeStruct((n, e_num), jnp.float32),
            jax.ShapeDtypeStruct((1, e_num), jnp.float32),
        ],
    )(x, gw1, gb1.reshape(1, d), gw2, gb2.reshape(1, e_num))

    xb = x.astype(jnp.bfloat16)
    w1_all = jnp.concatenate([ew1, aw1[None]], axis=0).astype(jnp.bfloat16)
    b1_all = jnp.concatenate([eb1, ab1[None]], axis=0).reshape(e_num + 1, 1, h_dim)
    w2_all = jnp.concatenate([ew2, aw2[None]], axis=0).astype(jnp.bfloat16)
    b2_all = jnp.concatenate([eb2, ab2[None]], axis=0).reshape(e_num + 1, 1, c_dim)

    logits, alpha = pl.pallas_call(
        functools.partial(_moe_kernel, n_exp=e_num),
        grid=(nt, e_num + 1),
        in_specs=[
            pl.BlockSpec((tn, d), lambda i, j: (i, 0)),
            pl.BlockSpec((1, d, h_dim), lambda i, j: (j, 0, 0)),
            pl.BlockSpec((1, 1, h_dim), lambda i, j: (j, 0, 0)),
            pl.BlockSpec((1, h_dim, c_dim), lambda i, j: (j, 0, 0)),
            pl.BlockSpec((1, 1, c_dim), lambda i, j: (j, 0, 0)),
            pl.BlockSpec((tn, e_num), lambda i, j: (i, 0)),
        ],
        out_specs=[
            pl.BlockSpec((tn, c_dim), lambda i, j: (i, 0)),
            pl.BlockSpec((tn, c_dim), lambda i, j: (i, 0)),
        ],
        out_shape=[
            jax.ShapeDtypeStruct((n, c_dim), jnp.float32),
            jax.ShapeDtypeStruct((n, c_dim), jnp.float32),
        ],
    )(xb, w1_all, b1_all, w2_all, b2_all, gate_weights)

    return (logits, alpha, gate_weights, load2d.reshape(e_num))
